# baseline (speedup 1.0000x reference)
import functools

import jax
import jax.numpy as jnp
from jax import lax
from jax.experimental import pallas as pl
from jax.experimental.pallas import tpu as pltpu
from jax.experimental.pallas import tpu_sc as plsc

_DECAY = 0.0001

_B = 16384
_D = 64
_NC = 2
_NS = 16
_NW = _NC * _NS
_BPW = _B // _NW
_IC = 128
_NG = _BPW // _IC

_mesh = plsc.VectorSubcoreMesh(core_axis_name="c", subcore_axis_name="s")


@functools.partial(
    pl.kernel,
    mesh=_mesh,
    out_type=jax.ShapeDtypeStruct((_B, 16), jnp.float32),
    compiler_params=pltpu.CompilerParams(use_tc_tiling_on_sc=False),
    scratch_types=[
        pltpu.VMEM((_NG, _IC), jnp.int32),
        pltpu.VMEM((_NG, _IC), jnp.int32),
        pltpu.VMEM((_NG, _IC), jnp.int32),
        pltpu.VMEM((_BPW, _D), jnp.float32),
        pltpu.VMEM((_BPW, _D), jnp.float32),
        pltpu.VMEM((_BPW, _D), jnp.float32),
        pltpu.VMEM((_BPW, 16), jnp.float32),
        pltpu.SemaphoreType.DMA,
    ],
)
def _sc_partials(users_emb, items_emb, users, pos_items, neg_items, out_hbm,
                 u_idx, p_idx, n_idx, u_rows, p_rows, n_rows, part, sem):
    wid = lax.axis_index("s") * _NC + lax.axis_index("c")
    base = wid * _BPW

    for j in range(_NG):
        off = base + j * _IC
        pltpu.sync_copy(users.at[pl.ds(off, _IC)], u_idx.at[j])
        pltpu.sync_copy(pos_items.at[pl.ds(off, _IC)], p_idx.at[j])
        pltpu.sync_copy(neg_items.at[pl.ds(off, _IC)], n_idx.at[j])

    copies = []
    for j in range(_NG):
        dst = pl.ds(j * _IC, _IC)
        copies.append(pltpu.async_copy(users_emb.at[u_idx.at[j]], u_rows.at[dst], sem))
        copies.append(pltpu.async_copy(items_emb.at[p_idx.at[j]], p_rows.at[dst], sem))
        copies.append(pltpu.async_copy(items_emb.at[n_idx.at[j]], n_rows.at[dst], sem))
    for c in copies:
        c.wait()

    def row_body(i, carry):
        t = jnp.zeros((16,), jnp.float32)
        for c in range(_D // 16):
            sl = pl.ds(c * 16, 16)
            t = t + u_rows[i, sl] * (p_rows[i, sl] - n_rows[i, sl])
        part[i, :] = t
        return carry

    lax.fori_loop(0, _BPW, row_body, 0)

    pltpu.sync_copy(part, out_hbm.at[pl.ds(base, _BPW)])


def _tc_loss_body(part_ref, cu_ref, cp_ref, cn_ref, out_ref):
    diff = jnp.sum(part_ref[...], axis=1)
    ls = jnp.minimum(diff, 0.0) - jnp.log1p(jnp.exp(-jnp.abs(diff)))
    loss = -jnp.sum(ls) / _B
    reg = 0.5 * (jnp.sum(cu_ref[...] * cu_ref[...])
                 + jnp.sum(cp_ref[...] * cp_ref[...])
                 + jnp.sum(cn_ref[...] * cn_ref[...])) / _B
    out_ref[...] = jnp.broadcast_to(loss + _DECAY * reg, (1, 1))


def kernel(users_emb, items_emb, users, pos_items, neg_items,
           current_user_emb, current_pos_item_emb, current_neg_item_emb):
    part = _sc_partials(users_emb, items_emb,
                        users.astype(jnp.int32),
                        pos_items.astype(jnp.int32),
                        neg_items.astype(jnp.int32))
    out = pl.pallas_call(
        _tc_loss_body,
        out_shape=jax.ShapeDtypeStruct((1, 1), jnp.float32),
    )(part, current_user_emb, current_pos_item_emb, current_neg_item_emb)
    return out[0, 0]


# speedup vs baseline: 1.9293x; 1.9293x over previous
import functools

import jax
import jax.numpy as jnp
from jax import lax
from jax.experimental import pallas as pl
from jax.experimental.pallas import tpu as pltpu
from jax.experimental.pallas import tpu_sc as plsc

_DECAY = 0.0001

_B = 16384
_D = 64
_NI = 1000000
_NU = 100000
_NW = 32
_BPW = _B // _NW

_ET_I = 245
_ET_U = 25
_LAST_I = (_NI - 1) // 128
_LAST_U = (_NU - 1) // 128
_CAP_I = 24
_CAP_U = 96
_ICH = 4096

_mesh = plsc.VectorSubcoreMesh(core_axis_name="c", subcore_axis_name="s")

_i32 = jnp.int32


def _iota16():
    return lax.iota(_i32, 16)


@functools.partial(
    pl.kernel,
    mesh=_mesh,
    compiler_params=pltpu.CompilerParams(needs_layout_passes=False),
    out_type=(
        jax.ShapeDtypeStruct((_B * _D,), jnp.float32),
        jax.ShapeDtypeStruct((_B * _D,), jnp.float32),
        jax.ShapeDtypeStruct((_B * _D,), jnp.float32),
    ),
    scratch_types=[
        pltpu.VMEM((_ICH,), _i32),
        pltpu.VMEM((256,), _i32),
        pltpu.VMEM((256,), _i32),
        pltpu.VMEM((32,), _i32),
        pltpu.VMEM((_ET_I * _CAP_I,), _i32),
        pltpu.VMEM((_ET_I * _CAP_I,), _i32),
        pltpu.VMEM((_ET_U * _CAP_U,), _i32),
        pltpu.VMEM((_D, 128), jnp.float32),
        pltpu.VMEM((_D, 128), jnp.float32),
        pltpu.VMEM((16, _D), jnp.float32),
        pltpu.VMEM((_D,), jnp.float32),
        pltpu.SemaphoreType.DMA,
        pltpu.SemaphoreType.DMA,
        pltpu.SemaphoreType.DMA,
    ],
)
def _sc_gather(users_t, items_t, users, pos_items, neg_items,
               gu, gp, gn,
               idx_buf, cp, cn, cu, bp, bn, bu,
               slab_a, slab_b, pack, drain, sem_a, sem_b, sem_s):
    wid = lax.axis_index("s") * 2 + lax.axis_index("c")
    it_lo = wid * _ET_I
    us_lo = wid * _ET_U

    z16 = jnp.zeros((16,), _i32)
    for q in range(16):
        cp[pl.ds(q * 16, 16)] = z16
        cn[pl.ds(q * 16, 16)] = z16
    cu[pl.ds(0, 16)] = z16
    cu[pl.ds(16, 16)] = z16

    def scan_array(arr, cnt_ref, buf_ref, lo_ent, n_ent, cap):
        def chunk_body(c, carry):
            pltpu.sync_copy(arr.at[pl.ds(c * _ICH, _ICH)], idx_buf)

            def vreg_body(g, carry2):
                e = idx_buf[pl.ds(g * 16, 16)]
                slot = c * _ICH + g * 16 + _iota16()
                m = (e >= lo_ent) & (e < lo_ent + n_ent)
                b = jnp.where(m, (e - lo_ent) >> 7, 0)
                rank, last = plsc.scan_count(b, m)
                cb = plsc.load_gather(cnt_ref, [b], mask=m)
                pos_ = b * cap + jnp.minimum(cb + rank - 1, cap - 1)
                pay = (e & 127) * 16384 + slot
                plsc.store_scatter(buf_ref, [pos_], pay, mask=m)
                plsc.store_scatter(cnt_ref, [b], cb + rank, mask=m & last)
                return carry2

            lax.fori_loop(0, _ICH // 16, vreg_body, 0)
            return carry

        lax.fori_loop(0, _B // _ICH, chunk_body, 0)

    scan_array(pos_items, cp, bp, it_lo * 128, _ET_I * 128, _CAP_I)
    scan_array(neg_items, cn, bn, it_lo * 128, _ET_I * 128, _CAP_I)
    scan_array(users, cu, bu, us_lo * 128, _ET_U * 128, _CAP_U)

    def slab_src(table, lo_et, t, last_et):
        off = jnp.minimum(lo_et + t, last_et) * 128
        return table.at[:, pl.ds(pl.multiple_of(off, 128), 128)]

    def extract(slab, cnt_ref, buf_ref, t, cap, out_flat):
        kv = plsc.load_gather(cnt_ref, [jnp.full((16,), t, _i32)])
        k = jnp.minimum(kv[0], cap)

        def group_body(g, carry):
            pay = buf_ref[pl.ds(t * cap + g * 16, 16)]
            valid = (g * 16 + _iota16()) < k
            lane = jnp.where(valid, pay >> 14, 0)
            slot = pay & 16383
            for f in range(_D):
                vals = plsc.load_gather(slab, [jnp.full((16,), f, _i32), lane],
                                        mask=valid)
                plsc.store_scatter(pack, [_iota16(), jnp.full((16,), f, _i32)],
                                   vals, mask=valid)
            for j in range(16):
                @pl.when(g * 16 + j < k)
                def _():
                    s = slot[j]
                    pltpu.async_copy(
                        pack.at[j],
                        out_flat.at[pl.ds(pl.multiple_of(s * _D, _D), _D)],
                        sem_s)
            return carry

        lax.fori_loop(0, (k + 15) // 16, group_body, 0)
        return k

    def fire(table, lo_et, t, last_et, slab, sem):
        pltpu.async_copy(slab_src(table, lo_et, t, last_et), slab, sem)

    def wait_slab(table, slab, sem):
        pltpu.make_async_copy(table.at[:, pl.ds(0, 128)], slab, sem).wait()

    def run_table(table, lo_et, n_et, last_et, jobs):
        fire(table, lo_et, 0, last_et, slab_a, sem_a)

        def etile_body(t, total):
            def on(slab, sem, other_slab, other_sem):
                res = total

                @pl.when(t + 1 < n_et)
                def _():
                    fire(table, lo_et, t + 1, last_et, other_slab, other_sem)

                wait_slab(table, slab, sem)
                for cnt_ref, buf_ref, cap, out_flat in jobs:
                    res_k = extract(slab, cnt_ref, buf_ref, t, cap, out_flat)
                    res = res + res_k
                return res

            return lax.cond(t % 2 == 0,
                            lambda: on(slab_a, sem_a, slab_b, sem_b),
                            lambda: on(slab_b, sem_b, slab_a, sem_a))

        return lax.fori_loop(0, n_et, etile_body, jnp.int32(0))

    tot_i = run_table(items_t, it_lo, _ET_I, _LAST_I,
                      [(cp, bp, _CAP_I, gp), (cn, bn, _CAP_I, gn)])
    tot_u = run_table(users_t, us_lo, _ET_U, _LAST_U,
                      [(cu, bu, _CAP_U, gu)])

    def drain_body(i, carry):
        pltpu.make_async_copy(gu.at[pl.ds(0, _D)], drain, sem_s).wait()
        return carry

    lax.fori_loop(0, tot_i + tot_u, drain_body, 0)


@functools.partial(
    pl.kernel,
    mesh=_mesh,
    compiler_params=pltpu.CompilerParams(needs_layout_passes=False),
    out_type=jax.ShapeDtypeStruct((_B, 16), jnp.float32),
    scratch_types=[
        pltpu.VMEM((128 * _D,), jnp.float32),
        pltpu.VMEM((128 * _D,), jnp.float32),
        pltpu.VMEM((128 * _D,), jnp.float32),
        pltpu.VMEM((_BPW, 16), jnp.float32),
    ],
)
def _sc_scores(gu, gp, gn, out_hbm, ub, pb, nb, part):
    wid = lax.axis_index("s") * 2 + lax.axis_index("c")
    base = wid * _BPW

    for h in range(_BPW // 128):
        off = (base + h * 128) * _D
        pltpu.sync_copy(gu.at[pl.ds(off, 128 * _D)], ub)
        pltpu.sync_copy(gp.at[pl.ds(off, 128 * _D)], pb)
        pltpu.sync_copy(gn.at[pl.ds(off, 128 * _D)], nb)

        def row_body(i, carry):
            t = jnp.zeros((16,), jnp.float32)
            for c in range(_D // 16):
                sl = pl.ds(i * _D + c * 16, 16)
                t = t + ub[sl] * (pb[sl] - nb[sl])
            part[h * 128 + i, :] = t
            return carry

        lax.fori_loop(0, 128, row_body, 0)

    pltpu.sync_copy(part, out_hbm.at[pl.ds(base, _BPW)])


def _tc_loss_body(part_ref, cu_ref, cp_ref, cn_ref, out_ref):
    diff = jnp.sum(part_ref[...], axis=1)
    ls = jnp.minimum(diff, 0.0) - jnp.log1p(jnp.exp(-jnp.abs(diff)))
    loss = -jnp.sum(ls) / _B
    reg = 0.5 * (jnp.sum(cu_ref[...] * cu_ref[...])
                 + jnp.sum(cp_ref[...] * cp_ref[...])
                 + jnp.sum(cn_ref[...] * cn_ref[...])) / _B
    out_ref[...] = jnp.broadcast_to(loss + _DECAY * reg, (1, 1))


def kernel(users_emb, items_emb, users, pos_items, neg_items,
           current_user_emb, current_pos_item_emb, current_neg_item_emb):
    gu, gp, gn = _sc_gather(users_emb.T, items_emb.T,
                            users.astype(_i32),
                            pos_items.astype(_i32),
                            neg_items.astype(_i32))
    part = _sc_scores(gu, gp, gn)
    out = pl.pallas_call(
        _tc_loss_body,
        out_shape=jax.ShapeDtypeStruct((1, 1), jnp.float32),
    )(part, current_user_emb.T, current_pos_item_emb.T, current_neg_item_emb.T)
    return out[0, 0]


# speedup vs baseline: 2.4605x; 1.2754x over previous
import functools

import jax
import jax.numpy as jnp
from jax import lax
from jax.experimental import pallas as pl
from jax.experimental.pallas import tpu as pltpu
from jax.experimental.pallas import tpu_sc as plsc

_DECAY = 0.0001

_B = 16384
_D = 64
_NI = 1000000
_NU = 100000
_NW = 32
_BPW = _B // _NW

_ET_I = 248
_ET_U = 28
_SW = 512
_NB_I = _ET_I * 128 // _SW
_NB_U = _ET_U * 128 // _SW
_PAD_I = 1000064
_PAD_U = 100096
_CAP_I = 40
_CAP_U = 176
_ICH = 4096

_mesh = plsc.VectorSubcoreMesh(core_axis_name="c", subcore_axis_name="s")

_i32 = jnp.int32


def _iota16():
    return lax.iota(_i32, 16)


@functools.partial(
    pl.kernel,
    mesh=_mesh,
    compiler_params=pltpu.CompilerParams(needs_layout_passes=False),
    out_type=(
        jax.ShapeDtypeStruct((_B * _D,), jnp.float32),
        jax.ShapeDtypeStruct((_B * _D,), jnp.float32),
        jax.ShapeDtypeStruct((_B * _D,), jnp.float32),
    ),
    scratch_types=[
        pltpu.VMEM((_ICH,), _i32),
        pltpu.VMEM((64,), _i32),
        pltpu.VMEM((64,), _i32),
        pltpu.VMEM((16,), _i32),
        pltpu.VMEM((_NB_I * _CAP_I,), _i32),
        pltpu.VMEM((_NB_I * _CAP_I,), _i32),
        pltpu.VMEM((_NB_U * _CAP_U,), _i32),
        pltpu.VMEM((_D, _SW), jnp.float32),
        pltpu.VMEM((_D, _SW), jnp.float32),
        pltpu.VMEM((16, _D), jnp.float32),
        pltpu.VMEM((_D,), jnp.float32),
        pltpu.SemaphoreType.DMA,
        pltpu.SemaphoreType.DMA,
        pltpu.SemaphoreType.DMA,
    ],
)
def _sc_gather(users_t, items_t, users, pos_items, neg_items,
               gu, gp, gn,
               idx_buf, cp, cn, cu, bp, bn, bu,
               slab_a, slab_b, pack, drain, sem_a, sem_b, sem_s):
    wid = lax.axis_index("s") * 2 + lax.axis_index("c")
    it_lo = wid * _ET_I * 128
    us_lo = wid * _ET_U * 128

    z16 = jnp.zeros((16,), _i32)
    for q in range(4):
        cp[pl.ds(q * 16, 16)] = z16
        cn[pl.ds(q * 16, 16)] = z16
    cu[pl.ds(0, 16)] = z16

    def scan_array(arr, cnt_ref, buf_ref, lo_ent, n_ent, cap):
        def chunk_body(c, carry):
            pltpu.sync_copy(arr.at[pl.ds(c * _ICH, _ICH)], idx_buf)

            def vreg_body(g, carry2):
                e = idx_buf[pl.ds(g * 16, 16)]
                slot = c * _ICH + g * 16 + _iota16()
                el = e - lo_ent
                m = (e >= lo_ent) & (e < lo_ent + n_ent)
                b = jnp.where(m, el >> 9, 0)
                rank, last = plsc.scan_count(b, m)
                cb = plsc.load_gather(cnt_ref, [b], mask=m)
                pos_ = b * cap + jnp.minimum(cb + rank - 1, cap - 1)
                pay = el * 16384 + slot
                plsc.store_scatter(buf_ref, [pos_], pay, mask=m)
                plsc.store_scatter(cnt_ref, [b], cb + rank, mask=m & last)
                return carry2

            lax.fori_loop(0, _ICH // 16, vreg_body, 0)
            return carry

        lax.fori_loop(0, _B // _ICH, chunk_body, 0)

    scan_array(pos_items, cp, bp, it_lo, _ET_I * 128, _CAP_I)
    scan_array(neg_items, cn, bn, it_lo, _ET_I * 128, _CAP_I)
    scan_array(users, cu, bu, us_lo, _ET_U * 128, _CAP_U)

    def slab_start(lo_ent, s, pad_end):
        return jnp.minimum(lo_ent + s * _SW, pad_end - _SW)

    def extract(slab, start_rel, cnt_ref, buf_ref, s, cap, out_flat):
        kv = plsc.load_gather(cnt_ref, [jnp.full((16,), s, _i32)])
        k = jnp.minimum(kv[0], cap)

        def group_body(g, carry):
            pay = buf_ref[pl.ds(s * cap + g * 16, 16)]
            valid = (g * 16 + _iota16()) < k
            el = pay >> 14
            lane = jnp.where(valid, el - start_rel, 0)
            slot = pay & 16383
            for f in range(_D):
                vals = plsc.load_gather(slab, [jnp.full((16,), f, _i32), lane],
                                        mask=valid)
                plsc.store_scatter(pack, [_iota16(), jnp.full((16,), f, _i32)],
                                   vals, mask=valid)
            for j in range(16):
                @pl.when(g * 16 + j < k)
                def _():
                    sl_ = slot[j]
                    pltpu.async_copy(
                        pack.at[j],
                        out_flat.at[pl.ds(pl.multiple_of(sl_ * _D, _D), _D)],
                        sem_s)
            return carry

        lax.fori_loop(0, (k + 15) // 16, group_body, 0)
        return k

    def fire(table, lo_ent, s, pad_end, slab, sem):
        off = pl.multiple_of(slab_start(lo_ent, s, pad_end), 128)
        pltpu.async_copy(table.at[:, pl.ds(off, _SW)], slab, sem)

    def wait_slab(table, slab, sem):
        pltpu.make_async_copy(table.at[:, pl.ds(0, _SW)], slab, sem).wait()

    def run_table(table, lo_ent, n_slab, pad_end, jobs):
        fire(table, lo_ent, 0, pad_end, slab_a, sem_a)

        def slab_body(s, total):
            def on(slab, sem, other_slab, other_sem):
                res = total

                @pl.when(s + 1 < n_slab)
                def _():
                    fire(table, lo_ent, s + 1, pad_end, other_slab, other_sem)

                wait_slab(table, slab, sem)
                start_rel = slab_start(lo_ent, s, pad_end) - lo_ent
                for cnt_ref, buf_ref, cap, out_flat in jobs:
                    res_k = extract(slab, start_rel, cnt_ref, buf_ref, s, cap,
                                    out_flat)
                    res = res + res_k
                return res

            return lax.cond(s % 2 == 0,
                            lambda: on(slab_a, sem_a, slab_b, sem_b),
                            lambda: on(slab_b, sem_b, slab_a, sem_a))

        return lax.fori_loop(0, n_slab, slab_body, jnp.int32(0))

    tot_i = run_table(items_t, it_lo, _NB_I, _PAD_I,
                      [(cp, bp, _CAP_I, gp), (cn, bn, _CAP_I, gn)])
    tot_u = run_table(users_t, us_lo, _NB_U, _PAD_U,
                      [(cu, bu, _CAP_U, gu)])

    def drain_body(i, carry):
        pltpu.make_async_copy(gu.at[pl.ds(0, _D)], drain, sem_s).wait()
        return carry

    lax.fori_loop(0, tot_i + tot_u, drain_body, 0)


@functools.partial(
    pl.kernel,
    mesh=_mesh,
    compiler_params=pltpu.CompilerParams(needs_layout_passes=False),
    out_type=jax.ShapeDtypeStruct((_B, 16), jnp.float32),
    scratch_types=[
        pltpu.VMEM((128 * _D,), jnp.float32),
        pltpu.VMEM((128 * _D,), jnp.float32),
        pltpu.VMEM((128 * _D,), jnp.float32),
        pltpu.VMEM((_BPW, 16), jnp.float32),
    ],
)
def _sc_scores(gu, gp, gn, out_hbm, ub, pb, nb, part):
    wid = lax.axis_index("s") * 2 + lax.axis_index("c")
    base = wid * _BPW

    for h in range(_BPW // 128):
        off = (base + h * 128) * _D
        pltpu.sync_copy(gu.at[pl.ds(off, 128 * _D)], ub)
        pltpu.sync_copy(gp.at[pl.ds(off, 128 * _D)], pb)
        pltpu.sync_copy(gn.at[pl.ds(off, 128 * _D)], nb)

        def row_body(i, carry):
            t = jnp.zeros((16,), jnp.float32)
            for c in range(_D // 16):
                sl = pl.ds(i * _D + c * 16, 16)
                t = t + ub[sl] * (pb[sl] - nb[sl])
            part[h * 128 + i, :] = t
            return carry

        lax.fori_loop(0, 128, row_body, 0)

    pltpu.sync_copy(part, out_hbm.at[pl.ds(base, _BPW)])


def _tc_loss_body(part_ref, cu_ref, cp_ref, cn_ref, out_ref):
    diff = jnp.sum(part_ref[...], axis=1)
    ls = jnp.minimum(diff, 0.0) - jnp.log1p(jnp.exp(-jnp.abs(diff)))
    loss = -jnp.sum(ls) / _B
    reg = 0.5 * (jnp.sum(cu_ref[...] * cu_ref[...])
                 + jnp.sum(cp_ref[...] * cp_ref[...])
                 + jnp.sum(cn_ref[...] * cn_ref[...])) / _B
    out_ref[...] = jnp.broadcast_to(loss + _DECAY * reg, (1, 1))


def kernel(users_emb, items_emb, users, pos_items, neg_items,
           current_user_emb, current_pos_item_emb, current_neg_item_emb):
    gu, gp, gn = _sc_gather(users_emb.T, items_emb.T,
                            users.astype(_i32),
                            pos_items.astype(_i32),
                            neg_items.astype(_i32))
    part = _sc_scores(gu, gp, gn)
    out = pl.pallas_call(
        _tc_loss_body,
        out_shape=jax.ShapeDtypeStruct((1, 1), jnp.float32),
    )(part, current_user_emb.T, current_pos_item_emb.T, current_neg_item_emb.T)
    return out[0, 0]
